# Initial kernel scaffold; baseline (speedup 1.0000x reference)
import jax
import jax.numpy as jnp
from jax.experimental import pallas as pl


def kernel(activations):
    raise NotImplementedError("write your pallas kernel here")


# speedup vs baseline: 29.5864x; 29.5864x over previous
import jax
import jax.numpy as jnp
from jax.experimental import pallas as pl

_N_PASS = 8


def _topk_mask_body(x_ref, o_ref):
    x = x_ref[0]
    t = jnp.full((1, x.shape[1]), jnp.inf, dtype=jnp.float32)
    neg = jnp.float32(-jnp.inf)
    for _ in range(_N_PASS):
        masked = jnp.where(x < t, x, neg)
        t = jnp.max(masked, axis=0, keepdims=True)
    o_ref[0] = (x >= t).astype(jnp.float32)


def kernel(activations):
    B, C, H, W = activations.shape
    P = H * W
    x = activations.reshape(B, C, P)
    out = pl.pallas_call(
        _topk_mask_body,
        grid=(B,),
        in_specs=[pl.BlockSpec((1, C, P), lambda i: (i, 0, 0))],
        out_specs=pl.BlockSpec((1, C, P), lambda i: (i, 0, 0)),
        out_shape=jax.ShapeDtypeStruct((B, C, P), jnp.float32),
    )(x)
    return out.reshape(B, C, H, W)
